# baseline (speedup 1.0000x reference)
import functools

import jax
import jax.numpy as jnp
from jax import lax
from jax.experimental import pallas as pl
from jax.experimental.pallas import tpu as pltpu
from jax.experimental.pallas import tpu_sc as plsc

_MAXREL = 16
_NIDX = 2 * _MAXREL + 1
_NPAD = 40


def _scores_body(q_ref, tab_ref, o_ref):
    o_ref[...] = jax.lax.dot_general(
        q_ref[...], tab_ref[...], (((1,), (1,)), ((), ())),
        preferred_element_type=jnp.float32)


def _make_sc_expand(B, H, Lq, Lk):
    mesh = plsc.VectorSubcoreMesh(core_axis_name="c", subcore_axis_name="s")
    n_worker = 32
    pairs_per_w = (B * Lq) // n_worker
    q_per_b = Lq
    w_per_b = q_per_b // pairs_per_w
    n_chunk = Lk // 16

    @functools.partial(
        pl.kernel,
        mesh=mesh,
        compiler_params=pltpu.CompilerParams(needs_layout_passes=False),
        out_type=jax.ShapeDtypeStruct((B, H, Lq, Lk), jnp.float32),
        scratch_types=[
            pltpu.VMEM((Lk,), jnp.int32),
            pltpu.VMEM((Lk,), jnp.int32),
            pltpu.VMEM((pairs_per_w * H * _NPAD,), jnp.float32),
            pltpu.VMEM((2, H, Lk), jnp.float32),
            pltpu.SemaphoreType.DMA((2,)),
        ],
    )
    def sc_expand(scores_hbm, tids_hbm, out_hbm, t_v, idx_v, s_v, o_v, sem):
        cid = lax.axis_index("c")
        sid = lax.axis_index("s")
        wid = sid * 2 + cid
        b = wid // w_per_b
        q0 = (wid % w_per_b) * pairs_per_w
        pltpu.sync_copy(tids_hbm.at[b], t_v)
        pltpu.sync_copy(
            scores_hbm.at[pl.ds((b * Lq + q0) * H * _NPAD,
                                pairs_per_w * H * _NPAD)], s_v)

        def do_q(j, carry):
            q = q0 + j
            p = j % 2

            @pl.when(j >= 2)
            def _wait_prev():
                pltpu.make_async_copy(
                    o_v.at[p], out_hbm.at[b, :, q, :], sem.at[p]).wait()

            tq = plsc.load_gather(t_v, [jnp.full((16,), q, jnp.int32)])
            base = j * (H * _NPAD) + _MAXREL

            def mk_idx(ci, carry2):
                tv = t_v[pl.ds(ci * 16, 16)]
                idx_v[pl.ds(ci * 16, 16)] = jnp.minimum(
                    jnp.maximum(tv - tq, -_MAXREL), _MAXREL)
                return carry2

            lax.fori_loop(0, n_chunk, mk_idx, 0, unroll=4)

            def do_chunk(ci, carry2):
                r = idx_v[pl.ds(ci * 16, 16)]
                vals = [plsc.load_gather(s_v, [r + (base + h * _NPAD)])
                        for h in range(H)]
                for h in range(H):
                    o_v[p, h, pl.ds(ci * 16, 16)] = vals[h]
                return carry2

            lax.fori_loop(0, n_chunk, do_chunk, 0, unroll=2)
            pltpu.async_copy(o_v.at[p], out_hbm.at[b, :, q, :], sem.at[p])
            return carry

        lax.fori_loop(0, pairs_per_w, do_q, 0)
        for jj in (pairs_per_w - 2, pairs_per_w - 1):
            pltpu.make_async_copy(
                o_v.at[jj % 2], out_hbm.at[b, :, q0 + jj, :],
                sem.at[jj % 2]).wait()

    return sc_expand


def kernel(query, time_ids, rel_table, k_len):
    B, H, Lq, Dh = query.shape
    Lk = time_ids.shape[1]
    start = k_len - Lk
    t = jax.lax.dynamic_slice_in_dim(time_ids, start, Lk, axis=1)
    t = t.astype(jnp.int32)

    q_flat = query.transpose(0, 2, 1, 3).reshape(B * Lq * H, Dh)
    tab_pad = jnp.zeros((_NPAD, Dh), jnp.float32).at[:_NIDX].set(rel_table)

    scores = pl.pallas_call(
        _scores_body,
        out_shape=jax.ShapeDtypeStruct((B * Lq * H, _NPAD), jnp.float32),
    )(q_flat, tab_pad)

    sc_expand = _make_sc_expand(B, H, Lq, Lk)
    return sc_expand(scores.reshape(-1), t)


# speedup vs baseline: 1.4739x; 1.4739x over previous
import functools

import jax
import jax.numpy as jnp
from jax import lax
from jax.experimental import pallas as pl
from jax.experimental.pallas import tpu as pltpu
from jax.experimental.pallas import tpu_sc as plsc

_MAXREL = 16
_NIDX = 2 * _MAXREL + 1
_NPAD = 40


def _scores_body(q_ref, tab_ref, o_ref):
    o_ref[...] = jax.lax.dot_general(
        q_ref[...], tab_ref[...], (((1,), (1,)), ((), ())),
        preferred_element_type=jnp.float32)


def _make_sc_expand(B, H, Lq, Lk):
    mesh = plsc.VectorSubcoreMesh(core_axis_name="c", subcore_axis_name="s")
    n_worker = 32
    pairs_per_w = (B * Lq) // n_worker
    q_per_b = Lq
    w_per_b = q_per_b // pairs_per_w
    n_chunk = Lk // 16

    @functools.partial(
        pl.kernel,
        mesh=mesh,
        compiler_params=pltpu.CompilerParams(needs_layout_passes=False),
        out_type=jax.ShapeDtypeStruct((B, H, Lq, Lk), jnp.float32),
        scratch_types=[
            pltpu.VMEM((Lk,), jnp.int32),
            pltpu.VMEM((Lk,), jnp.int32),
            pltpu.VMEM((pairs_per_w * H * _NPAD,), jnp.float32),
            pltpu.VMEM((2, H, Lk), jnp.float32),
            pltpu.SemaphoreType.DMA((2,)),
        ],
    )
    def sc_expand(scores_hbm, tids_hbm, out_hbm, t_v, idx_v, s_v, o_v, sem):
        cid = lax.axis_index("c")
        sid = lax.axis_index("s")
        wid = sid * 2 + cid
        b = wid // w_per_b
        q0 = (wid % w_per_b) * pairs_per_w
        pltpu.sync_copy(tids_hbm.at[b], t_v)
        pltpu.sync_copy(
            scores_hbm.at[pl.ds((b * Lq + q0) * H * _NPAD,
                                pairs_per_w * H * _NPAD)], s_v)

        def do_q(j, carry):
            q = q0 + j
            p = j % 2

            @pl.when(j >= 2)
            def _wait_prev():
                pltpu.make_async_copy(
                    o_v.at[p], out_hbm.at[b, :, q, :], sem.at[p]).wait()

            tq = plsc.load_gather(t_v, [jnp.full((16,), q, jnp.int32)])
            base = j * (H * _NPAD) + _MAXREL

            @plsc.parallel_loop(0, n_chunk, unroll=2)
            def do_chunk(ci):
                tv = t_v[pl.ds(ci * 16, 16)]
                r = jnp.minimum(jnp.maximum(tv - tq, -_MAXREL), _MAXREL)
                vals = [plsc.load_gather(s_v, [r + (base + h * _NPAD)])
                        for h in range(H)]
                for h in range(H):
                    o_v[p, h, pl.ds(ci * 16, 16)] = vals[h]
            pltpu.async_copy(o_v.at[p], out_hbm.at[b, :, q, :], sem.at[p])
            return carry

        lax.fori_loop(0, pairs_per_w, do_q, 0)
        for jj in (pairs_per_w - 2, pairs_per_w - 1):
            pltpu.make_async_copy(
                o_v.at[jj % 2], out_hbm.at[b, :, q0 + jj, :],
                sem.at[jj % 2]).wait()

    return sc_expand


def kernel(query, time_ids, rel_table, k_len):
    B, H, Lq, Dh = query.shape
    Lk = time_ids.shape[1]
    start = k_len - Lk
    t = jax.lax.dynamic_slice_in_dim(time_ids, start, Lk, axis=1)
    t = t.astype(jnp.int32)

    q_flat = query.transpose(0, 2, 1, 3).reshape(B * Lq * H, Dh)
    tab_pad = jnp.zeros((_NPAD, Dh), jnp.float32).at[:_NIDX].set(rel_table)

    scores = pl.pallas_call(
        _scores_body,
        out_shape=jax.ShapeDtypeStruct((B * Lq * H, _NPAD), jnp.float32),
    )(q_flat, tab_pad)

    sc_expand = _make_sc_expand(B, H, Lq, Lk)
    return sc_expand(scores.reshape(-1), t)
